# baseline (speedup 1.0000x reference)
import functools

import jax
import jax.numpy as jnp
from jax import lax
from jax.experimental import pallas as pl
from jax.experimental.pallas import tpu as pltpu
from jax.experimental.pallas import tpu_sc as plsc

N = 10000
NP = 10240
E = 320000
NC = 2
NS = 16
NW = NC * NS
EPT = 10240
EC = EPT // 128
EPAD = NW * EPT
D = 128
DOUT = 16
RB = 1024


def _mesh():
    return plsc.VectorSubcoreMesh(core_axis_name="c", subcore_axis_name="s")


def _deg_body(srcs, dsts, znp, out, idx_v, acc_v):
    c = lax.axis_index("c")
    s = lax.axis_index("s")
    w = c * NS + s
    ones = jnp.ones((16,), jnp.float32)
    for r in range(3):
        for side, ei in ((0, srcs), (1, dsts)):
            pltpu.sync_copy(ei.at[r, w], idx_v)
            pltpu.sync_copy(znp, acc_v)

            @pl.loop(0, EC)
            def _(j):
                for k in range(8):
                    idx16 = idx_v[pl.ds(j * 128 + k * 16, 16)]
                    plsc.addupdate_scatter(acc_v, [idx16], ones)

            pltpu.sync_copy(acc_v, out.at[w, 2 * r + side])


_deg_kernel = functools.partial(
    pl.kernel,
    out_type=jax.ShapeDtypeStruct((NW, 6, NP), jnp.float32),
    mesh=_mesh(),
    compiler_params=pltpu.CompilerParams(needs_layout_passes=False),
    scratch_types=[
        pltpu.VMEM((EPT,), jnp.int32),
        pltpu.VMEM((NP,), jnp.float32),
    ],
)(_deg_body)


EC2 = 2 * EC
K = 3
H = 1


def _edge_pipeline(tabref, acc_sp, sidx, didx, rowbufs, gsem, ssem, nchunks):
    def gfire(i, b):
        pltpu.async_copy(tabref.at[sidx.at[i]], rowbufs.at[b], gsem.at[b])

    def gwait(i, b):
        pltpu.make_async_copy(tabref.at[sidx.at[i]], rowbufs.at[b],
                              gsem.at[b]).wait()

    def sfire(i, b):
        pltpu.async_copy(rowbufs.at[b], acc_sp.at[didx.at[i]], ssem.at[b],
                         add=True)

    def swait(i, b):
        pltpu.make_async_copy(rowbufs.at[b], acc_sp.at[didx.at[i]],
                              ssem.at[b]).wait()

    def steady(i, bu):
        gwait(i, bu)
        sfire(i, bu)
        b2 = (bu - H) % K
        swait(i - H, b2)
        gfire(i - H + K, b2)

    L = ((nchunks - 2 * H) // K) * K
    for i in range(K):
        gfire(i, i)
    for i in range(H):
        gwait(i, i)
        sfire(i, i)

    @pl.loop(H, H + L, step=K)
    def _(j):
        for u in range(K):
            steady(j + u, (H + u) % K)

    for i in range(H + L, nchunks):
        bu = i % K
        gwait(i, bu)
        sfire(i, bu)
        b2 = (bu - H) % K
        swait(i - H, b2)
        if i - H + K < nchunks:
            gfire(i - H + K, b2)
    for i in range(nchunks - H, nchunks):
        swait(i, i % K)


def _agg128_body(src, dst, table, zeros_hbm, out, sidx, didx, rowbufs,
                 acc_sp, tab_sp, gsem, ssem):
    c = lax.axis_index("c")
    s = lax.axis_index("s")
    rows = NP // NS
    for r in range(3):
        for t in range(rows // 128):
            pltpu.sync_copy(zeros_hbm,
                            acc_sp.at[pl.ds(s * rows + t * 128, 128)])
        pltpu.sync_copy(table.at[r, c, pl.ds(s * rows, rows)],
                        tab_sp.at[pl.ds(s * rows, rows)])
        pltpu.sync_copy(src.at[r, s, pl.ds(0, EC)], sidx)
        pltpu.sync_copy(dst.at[r, s, pl.ds(0, EC)], didx)
        plsc.subcore_barrier()
        for phase in range(EC2 // EC):
            if phase > 0:
                pltpu.sync_copy(src.at[r, s, pl.ds(phase * EC, EC)], sidx)
                pltpu.sync_copy(dst.at[r, s, pl.ds(phase * EC, EC)], didx)
            _edge_pipeline(tab_sp, acc_sp, sidx, didx, rowbufs, gsem, ssem,
                           EC)
        plsc.subcore_barrier()
        pltpu.sync_copy(acc_sp.at[pl.ds(s * rows, rows)],
                        out.at[r, c, pl.ds(s * rows, rows)])


_agg128 = functools.partial(
    pl.kernel,
    out_type=jax.ShapeDtypeStruct((3, 2, NP, 64), jnp.float32),
    mesh=_mesh(),
    compiler_params=pltpu.CompilerParams(needs_layout_passes=False,
                                         use_tc_tiling_on_sc=False),
    scratch_types=[
        pltpu.VMEM((EC, 128), jnp.int32),
        pltpu.VMEM((EC, 128), jnp.int32),
        pltpu.VMEM((K, 128, 64), jnp.float32),
        pltpu.VMEM_SHARED((NP, 64), jnp.float32),
        pltpu.VMEM_SHARED((NP, 64), jnp.float32),
        pltpu.SemaphoreType.DMA((K,)),
        pltpu.SemaphoreType.DMA((K,)),
    ],
)(_agg128_body)


def _agg16_body(src, dst, table, zeros_hbm, out, sidx, didx, rowbufs,
                acc_sp, tab_sp, gsem, ssem):
    c = lax.axis_index("c")
    s = lax.axis_index("s")
    rows = NP // NS
    for r in range(3):
        pltpu.sync_copy(src.at[r, s, pl.ds(c * EC, EC)], sidx)
        pltpu.sync_copy(dst.at[r, s, pl.ds(c * EC, EC)], didx)
        for t in range(rows // 128):
            pltpu.sync_copy(zeros_hbm,
                            acc_sp.at[pl.ds(s * rows + t * 128, 128)])
        pltpu.sync_copy(table.at[r, pl.ds(s * rows, rows)],
                        tab_sp.at[pl.ds(s * rows, rows)])
        plsc.subcore_barrier()
        _edge_pipeline(tab_sp, acc_sp, sidx, didx, rowbufs, gsem, ssem, EC)
        plsc.subcore_barrier()
        pltpu.sync_copy(acc_sp.at[pl.ds(s * rows, rows)],
                        out.at[r, c, pl.ds(s * rows, rows)])


_agg16 = functools.partial(
    pl.kernel,
    out_type=jax.ShapeDtypeStruct((3, 2, NP, DOUT), jnp.float32),
    mesh=_mesh(),
    compiler_params=pltpu.CompilerParams(needs_layout_passes=False,
                                         use_tc_tiling_on_sc=False),
    scratch_types=[
        pltpu.VMEM((EC, 128), jnp.int32),
        pltpu.VMEM((EC, 128), jnp.int32),
        pltpu.VMEM((K, 128, DOUT), jnp.float32),
        pltpu.VMEM_SHARED((NP, DOUT), jnp.float32),
        pltpu.VMEM_SHARED((NP, DOUT), jnp.float32),
        pltpu.SemaphoreType.DMA((K,)),
        pltpu.SemaphoreType.DMA((K,)),
    ],
)(_agg16_body)


def _write_split(t, r, res):
    t[r, 0] = res[:, :64]
    t[r, 1] = res[:, 64:]


def _pre_body(counts, x, w, scales, t):
    cnt = jnp.sum(counts[...], axis=0)
    sc = lax.rsqrt(jnp.maximum(cnt, 1.0))
    scales[...] = sc
    for r in range(3):
        xs = x[...] * sc[2 * r][:, None]
        _write_split(t, r, jnp.dot(xs, w[r], preferred_element_type=jnp.float32))


def _tc_pre(counts, x, w):
    grid = NP // RB
    return pl.pallas_call(
        _pre_body,
        grid=(grid,),
        in_specs=[
            pl.BlockSpec((NW, 6, RB), lambda i: (0, 0, i)),
            pl.BlockSpec((RB, D), lambda i: (i, 0)),
            pl.BlockSpec((3, D, D), lambda i: (0, 0, 0)),
        ],
        out_specs=[
            pl.BlockSpec((6, RB), lambda i: (0, i)),
            pl.BlockSpec((3, 2, RB, 64), lambda i: (0, 0, i, 0)),
        ],
        out_shape=[
            jax.ShapeDtypeStruct((6, NP), jnp.float32),
            jax.ShapeDtypeStruct((3, 2, NP, 64), jnp.float32),
        ],
    )(counts, x, w)


def _mid_body(dn, p, scales, b, w, t):
    sc = scales[...]
    h = None
    for r in range(3):
        agg = jnp.concatenate([p[r, 0], p[r, 1]], axis=1)
        v = agg * sc[2 * r + 1][:, None] + b[r][None, :]
        v = jnp.maximum(v, 0.0)
        h = v if h is None else h + v
    h = h * (1.0 / 3.0)
    for r in range(3):
        res = jnp.dot(h * sc[2 * r][:, None], w[r],
                      preferred_element_type=jnp.float32)
        if dn == D:
            _write_split(t, r, res)
        else:
            t[r] = res


def _tc_mid(p, scales, b, w, dn):
    grid = NP // RB
    if dn == D:
        out_spec = pl.BlockSpec((3, 2, RB, 64), lambda i: (0, 0, i, 0))
        out_shape = jax.ShapeDtypeStruct((3, 2, NP, 64), jnp.float32)
    else:
        out_spec = pl.BlockSpec((3, RB, dn), lambda i: (0, i, 0))
        out_shape = jax.ShapeDtypeStruct((3, NP, dn), jnp.float32)
    return pl.pallas_call(
        functools.partial(_mid_body, dn),
        grid=(grid,),
        in_specs=[
            pl.BlockSpec((3, 2, RB, 64), lambda i: (0, 0, i, 0)),
            pl.BlockSpec((6, RB), lambda i: (0, i)),
            pl.BlockSpec((3, D), lambda i: (0, 0)),
            pl.BlockSpec((3, D, dn), lambda i: (0, 0, 0)),
        ],
        out_specs=out_spec,
        out_shape=out_shape,
    )(p, scales, b, w)


def _fin_body(p, scales, b, out):
    sc = scales[...]
    h = None
    for r in range(3):
        v = (p[r, 0] + p[r, 1]) * sc[2 * r + 1][:, None] + b[r][None, :]
        h = v if h is None else h + v
    out[...] = h * (1.0 / 3.0)


def _tc_fin(p, scales, b):
    grid = NP // RB
    return pl.pallas_call(
        _fin_body,
        grid=(grid,),
        in_specs=[
            pl.BlockSpec((3, 2, RB, DOUT), lambda i: (0, 0, i, 0)),
            pl.BlockSpec((6, RB), lambda i: (0, i)),
            pl.BlockSpec((3, DOUT), lambda i: (0, 0)),
        ],
        out_specs=pl.BlockSpec((RB, DOUT), lambda i: (i, 0)),
        out_shape=jax.ShapeDtypeStruct((NP, DOUT), jnp.float32),
    )(p, scales, b)


def kernel(x, edge_index_r0, edge_index_r1, edge_index_r2,
           W_l0_r0, b_l0_r0, W_l0_r1, b_l0_r1, W_l0_r2, b_l0_r2,
           W_l1_r0, b_l1_r0, W_l1_r1, b_l1_r1, W_l1_r2, b_l1_r2,
           W_l2_r0, b_l2_r0, W_l2_r1, b_l2_r1, W_l2_r2, b_l2_r2):
    eis = [edge_index_r0, edge_index_r1, edge_index_r2]
    srcs2, dsts2 = [], []
    for ei in eis:
        srcs2.append(jnp.pad(ei[0], (0, EPAD - E),
                             constant_values=N).reshape(NS, EC2, 128))
        dsts2.append(jnp.pad(ei[1], (0, EPAD - E),
                             constant_values=N).reshape(NS, EC2, 128))
    xp = jnp.pad(x, ((0, NP - N), (0, 0)))
    z64 = jnp.zeros((128, 64), jnp.float32)
    z16 = jnp.zeros((128, DOUT), jnp.float32)
    znp = jnp.zeros((NP,), jnp.float32)

    W0 = jnp.stack([W_l0_r0, W_l0_r1, W_l0_r2])
    W1 = jnp.stack([W_l1_r0, W_l1_r1, W_l1_r2])
    W2 = jnp.stack([W_l2_r0, W_l2_r1, W_l2_r2])
    B0 = jnp.stack([b_l0_r0, b_l0_r1, b_l0_r2])
    B1 = jnp.stack([b_l1_r0, b_l1_r1, b_l1_r2])
    B2 = jnp.stack([b_l2_r0, b_l2_r1, b_l2_r2])

    src3 = jnp.stack(srcs2)
    dst3 = jnp.stack(dsts2)
    counts = _deg_kernel(src3.reshape(3, NW, EPT), dst3.reshape(3, NW, EPT),
                         znp)

    scales, tables = _tc_pre(counts, xp, W0)
    aggs = _agg128(src3, dst3, tables, z64)
    tables = _tc_mid(aggs, scales, B0, W1, D)
    aggs = _agg128(src3, dst3, tables, z64)
    tables = _tc_mid(aggs, scales, B1, W2, DOUT)
    aggs = _agg16(src3, dst3, tables, z16)
    out = _tc_fin(aggs, scales, B2)
    return out[:N]


# speedup vs baseline: 1.0219x; 1.0219x over previous
import functools

import jax
import jax.numpy as jnp
from jax import lax
from jax.experimental import pallas as pl
from jax.experimental.pallas import tpu as pltpu
from jax.experimental.pallas import tpu_sc as plsc

N = 10000
NP = 10240
E = 320000
NC = 2
NS = 16
NW = NC * NS
EPT = 10240
EC = EPT // 128
EPAD = NW * EPT
D = 128
DOUT = 16
RB = 1024


def _mesh():
    return plsc.VectorSubcoreMesh(core_axis_name="c", subcore_axis_name="s")


def _deg_body(s0, d0, s1, d1, s2, d2, znp, out, idx_v, acc_v):
    c = lax.axis_index("c")
    s = lax.axis_index("s")
    w = c * NS + s
    ones = jnp.ones((16,), jnp.float32)
    for side, ei in enumerate((s0, d0, s1, d1, s2, d2)):
        pltpu.sync_copy(ei.at[w], idx_v)
        pltpu.sync_copy(znp, acc_v)

        @pl.loop(0, EC)
        def _(j):
            for k in range(8):
                idx16 = idx_v[pl.ds(j * 128 + k * 16, 16)]
                plsc.addupdate_scatter(acc_v, [idx16], ones)

        pltpu.sync_copy(acc_v, out.at[w, side])


_deg_kernel = functools.partial(
    pl.kernel,
    out_type=jax.ShapeDtypeStruct((NW, 6, NP), jnp.float32),
    mesh=_mesh(),
    compiler_params=pltpu.CompilerParams(needs_layout_passes=False),
    scratch_types=[
        pltpu.VMEM((EPT,), jnp.int32),
        pltpu.VMEM((NP,), jnp.float32),
    ],
)(_deg_body)


EC2 = 2 * EC
K = 3
H = 1


def _edge_pipeline(tabref, acc_sp, sidx, didx, rowbufs, gsem, ssem, nchunks):
    def gfire(i, b):
        pltpu.async_copy(tabref.at[sidx.at[i]], rowbufs.at[b], gsem.at[b])

    def gwait(i, b):
        pltpu.make_async_copy(tabref.at[sidx.at[i]], rowbufs.at[b],
                              gsem.at[b]).wait()

    def sfire(i, b):
        pltpu.async_copy(rowbufs.at[b], acc_sp.at[didx.at[i]], ssem.at[b],
                         add=True)

    def swait(i, b):
        pltpu.make_async_copy(rowbufs.at[b], acc_sp.at[didx.at[i]],
                              ssem.at[b]).wait()

    def steady(i, bu):
        gwait(i, bu)
        sfire(i, bu)
        b2 = (bu - H) % K
        swait(i - H, b2)
        gfire(i - H + K, b2)

    L = ((nchunks - 2 * H) // K) * K
    for i in range(K):
        gfire(i, i)
    for i in range(H):
        gwait(i, i)
        sfire(i, i)

    @pl.loop(H, H + L, step=K)
    def _(j):
        for u in range(K):
            steady(j + u, (H + u) % K)

    for i in range(H + L, nchunks):
        bu = i % K
        gwait(i, bu)
        sfire(i, bu)
        b2 = (bu - H) % K
        swait(i - H, b2)
        if i - H + K < nchunks:
            gfire(i - H + K, b2)
    for i in range(nchunks - H, nchunks):
        swait(i, i % K)


def _agg128_body(src, dst, table, zeros_hbm, out, sidx, didx, rowbufs,
                 acc_sp, tab_sp, gsem, ssem):
    c = lax.axis_index("c")
    s = lax.axis_index("s")
    rows = NP // NS
    for r in range(3):
        for t in range(rows // 128):
            pltpu.sync_copy(zeros_hbm,
                            acc_sp.at[pl.ds(s * rows + t * 128, 128)])
        pltpu.sync_copy(table.at[r, c, pl.ds(s * rows, rows)],
                        tab_sp.at[pl.ds(s * rows, rows)])
        pltpu.sync_copy(src.at[r, s, pl.ds(0, EC)], sidx)
        pltpu.sync_copy(dst.at[r, s, pl.ds(0, EC)], didx)
        plsc.subcore_barrier()
        for phase in range(EC2 // EC):
            if phase > 0:
                pltpu.sync_copy(src.at[r, s, pl.ds(phase * EC, EC)], sidx)
                pltpu.sync_copy(dst.at[r, s, pl.ds(phase * EC, EC)], didx)
            _edge_pipeline(tab_sp, acc_sp, sidx, didx, rowbufs, gsem, ssem,
                           EC)
        plsc.subcore_barrier()
        pltpu.sync_copy(acc_sp.at[pl.ds(s * rows, rows)],
                        out.at[r, c, pl.ds(s * rows, rows)])


_agg128 = functools.partial(
    pl.kernel,
    out_type=jax.ShapeDtypeStruct((3, 2, NP, 64), jnp.float32),
    mesh=_mesh(),
    compiler_params=pltpu.CompilerParams(needs_layout_passes=False,
                                         use_tc_tiling_on_sc=False),
    scratch_types=[
        pltpu.VMEM((EC, 128), jnp.int32),
        pltpu.VMEM((EC, 128), jnp.int32),
        pltpu.VMEM((K, 128, 64), jnp.float32),
        pltpu.VMEM_SHARED((NP, 64), jnp.float32),
        pltpu.VMEM_SHARED((NP, 64), jnp.float32),
        pltpu.SemaphoreType.DMA((K,)),
        pltpu.SemaphoreType.DMA((K,)),
    ],
)(_agg128_body)


def _agg16_body(src, dst, table, zeros_hbm, out, sidx, didx, rowbufs,
                acc_sp, tab_sp, gsem, ssem):
    c = lax.axis_index("c")
    s = lax.axis_index("s")
    w = c * NS + s
    rows = NP // NS
    for r in range(3):
        pltpu.sync_copy(src.at[r, w], sidx)
        pltpu.sync_copy(dst.at[r, w], didx)
        for t in range(rows // 128):
            pltpu.sync_copy(zeros_hbm,
                            acc_sp.at[pl.ds(s * rows + t * 128, 128)])
        pltpu.sync_copy(table.at[r, pl.ds(s * rows, rows)],
                        tab_sp.at[pl.ds(s * rows, rows)])
        plsc.subcore_barrier()
        _edge_pipeline(tab_sp, acc_sp, sidx, didx, rowbufs, gsem, ssem, EC)
        plsc.subcore_barrier()
        pltpu.sync_copy(acc_sp.at[pl.ds(s * rows, rows)],
                        out.at[r, c, pl.ds(s * rows, rows)])


_agg16 = functools.partial(
    pl.kernel,
    out_type=jax.ShapeDtypeStruct((3, 2, NP, DOUT), jnp.float32),
    mesh=_mesh(),
    compiler_params=pltpu.CompilerParams(needs_layout_passes=False,
                                         use_tc_tiling_on_sc=False),
    scratch_types=[
        pltpu.VMEM((EC, 128), jnp.int32),
        pltpu.VMEM((EC, 128), jnp.int32),
        pltpu.VMEM((K, 128, DOUT), jnp.float32),
        pltpu.VMEM_SHARED((NP, DOUT), jnp.float32),
        pltpu.VMEM_SHARED((NP, DOUT), jnp.float32),
        pltpu.SemaphoreType.DMA((K,)),
        pltpu.SemaphoreType.DMA((K,)),
    ],
)(_agg16_body)


def _write_split(t, r, res):
    t[r, 0] = res[:, :64]
    t[r, 1] = res[:, 64:]


def _pre_body(counts, x, w, scales, t):
    cnt = jnp.sum(counts[...], axis=0)
    sc = lax.rsqrt(jnp.maximum(cnt, 1.0))
    scales[...] = sc
    for r in range(3):
        xs = x[...] * sc[2 * r][:, None]
        _write_split(t, r, jnp.dot(xs, w[r], preferred_element_type=jnp.float32))


def _tc_pre(counts, x, w):
    grid = NP // RB
    return pl.pallas_call(
        _pre_body,
        grid=(grid,),
        in_specs=[
            pl.BlockSpec((NW, 6, RB), lambda i: (0, 0, i)),
            pl.BlockSpec((RB, D), lambda i: (i, 0)),
            pl.BlockSpec((3, D, D), lambda i: (0, 0, 0)),
        ],
        out_specs=[
            pl.BlockSpec((6, RB), lambda i: (0, i)),
            pl.BlockSpec((3, 2, RB, 64), lambda i: (0, 0, i, 0)),
        ],
        out_shape=[
            jax.ShapeDtypeStruct((6, NP), jnp.float32),
            jax.ShapeDtypeStruct((3, 2, NP, 64), jnp.float32),
        ],
    )(counts, x, w)


def _mid_body(dn, p, scales, b, w, t):
    sc = scales[...]
    h = None
    for r in range(3):
        agg = jnp.concatenate([p[r, 0], p[r, 1]], axis=1)
        v = agg * sc[2 * r + 1][:, None] + b[r][None, :]
        v = jnp.maximum(v, 0.0)
        h = v if h is None else h + v
    h = h * (1.0 / 3.0)
    for r in range(3):
        res = jnp.dot(h * sc[2 * r][:, None], w[r],
                      preferred_element_type=jnp.float32)
        if dn == D:
            _write_split(t, r, res)
        else:
            t[r] = res


def _tc_mid(p, scales, b, w, dn):
    grid = NP // RB
    if dn == D:
        out_spec = pl.BlockSpec((3, 2, RB, 64), lambda i: (0, 0, i, 0))
        out_shape = jax.ShapeDtypeStruct((3, 2, NP, 64), jnp.float32)
    else:
        out_spec = pl.BlockSpec((3, RB, dn), lambda i: (0, i, 0))
        out_shape = jax.ShapeDtypeStruct((3, NP, dn), jnp.float32)
    return pl.pallas_call(
        functools.partial(_mid_body, dn),
        grid=(grid,),
        in_specs=[
            pl.BlockSpec((3, 2, RB, 64), lambda i: (0, 0, i, 0)),
            pl.BlockSpec((6, RB), lambda i: (0, i)),
            pl.BlockSpec((3, D), lambda i: (0, 0)),
            pl.BlockSpec((3, D, dn), lambda i: (0, 0, 0)),
        ],
        out_specs=out_spec,
        out_shape=out_shape,
    )(p, scales, b, w)


def _fin_body(p, scales, b, out):
    sc = scales[...]
    h = None
    for r in range(3):
        v = (p[r, 0] + p[r, 1]) * sc[2 * r + 1][:, None] + b[r][None, :]
        h = v if h is None else h + v
    out[...] = h * (1.0 / 3.0)


def _tc_fin(p, scales, b):
    grid = NP // RB
    return pl.pallas_call(
        _fin_body,
        grid=(grid,),
        in_specs=[
            pl.BlockSpec((3, 2, RB, DOUT), lambda i: (0, 0, i, 0)),
            pl.BlockSpec((6, RB), lambda i: (0, i)),
            pl.BlockSpec((3, DOUT), lambda i: (0, 0)),
        ],
        out_specs=pl.BlockSpec((RB, DOUT), lambda i: (i, 0)),
        out_shape=jax.ShapeDtypeStruct((NP, DOUT), jnp.float32),
    )(p, scales, b)


def kernel(x, edge_index_r0, edge_index_r1, edge_index_r2,
           W_l0_r0, b_l0_r0, W_l0_r1, b_l0_r1, W_l0_r2, b_l0_r2,
           W_l1_r0, b_l1_r0, W_l1_r1, b_l1_r1, W_l1_r2, b_l1_r2,
           W_l2_r0, b_l2_r0, W_l2_r1, b_l2_r1, W_l2_r2, b_l2_r2):
    eis = [edge_index_r0, edge_index_r1, edge_index_r2]
    srcs, dsts, srcs2, dsts2, srcs_f, dsts_f = [], [], [], [], [], []
    for ei in eis:
        sp = jnp.pad(ei[0], (0, EPAD - E), constant_values=N)
        dp = jnp.pad(ei[1], (0, EPAD - E), constant_values=N)
        srcs.append(sp.reshape(NW, EC, 128))
        dsts.append(dp.reshape(NW, EC, 128))
        srcs2.append(sp.reshape(NS, EC2, 128))
        dsts2.append(dp.reshape(NS, EC2, 128))
        srcs_f.append(sp.reshape(NW, EPT))
        dsts_f.append(dp.reshape(NW, EPT))
    xp = jnp.pad(x, ((0, NP - N), (0, 0)))
    z64 = jnp.zeros((128, 64), jnp.float32)
    z16 = jnp.zeros((128, DOUT), jnp.float32)
    znp = jnp.zeros((NP,), jnp.float32)

    W0 = jnp.stack([W_l0_r0, W_l0_r1, W_l0_r2])
    W1 = jnp.stack([W_l1_r0, W_l1_r1, W_l1_r2])
    W2 = jnp.stack([W_l2_r0, W_l2_r1, W_l2_r2])
    B0 = jnp.stack([b_l0_r0, b_l0_r1, b_l0_r2])
    B1 = jnp.stack([b_l1_r0, b_l1_r1, b_l1_r2])
    B2 = jnp.stack([b_l2_r0, b_l2_r1, b_l2_r2])

    counts = _deg_kernel(srcs_f[0], dsts_f[0], srcs_f[1], dsts_f[1],
                         srcs_f[2], dsts_f[2], znp)
    src3 = jnp.stack(srcs2)
    dst3 = jnp.stack(dsts2)
    src3w = jnp.stack(srcs)
    dst3w = jnp.stack(dsts)

    scales, tables = _tc_pre(counts, xp, W0)
    aggs = _agg128(src3, dst3, tables, z64)
    tables = _tc_mid(aggs, scales, B0, W1, D)
    aggs = _agg128(src3, dst3, tables, z64)
    tables = _tc_mid(aggs, scales, B1, W2, DOUT)
    aggs = _agg16(src3w, dst3w, tables, z16)
    out = _tc_fin(aggs, scales, B2)
    return out[:N]
